# Initial kernel scaffold; baseline (speedup 1.0000x reference)
import jax
import jax.numpy as jnp
from jax.experimental import pallas as pl


def kernel(x, edge_index, batch, W1, a_src1, a_dst1, b1, g1, be1, W2, a_src2, a_dst2, b2, g2, be2, W3, a_src3, a_dst3, b3, g3, be3, Wp1, bp1, Wp2, bp2):
    raise NotImplementedError("write your pallas kernel here")


# speedup vs baseline: 29.2786x; 29.2786x over previous
import functools

import jax
import jax.numpy as jnp
from jax import lax
from jax.experimental import pallas as pl
from jax.experimental.pallas import tpu as pltpu
from jax.experimental.pallas import tpu_sc as plsc

N = 50000
E = 800000
G = 64
H = 4
CG = 16
D = 4 * CG

NB = 5000
NBLK = N // NB
NPT = N // 16
EPT = E // 16


def _head_body(ng, have_bn):
    def body(*refs):
        if have_bn:
            (p_ref, b_ref, st_ref, g_ref, be_ref, w_ref, vs_ref, vd_ref,
             h_ref, es_ref, ed_ref, bt_ref, mx_scr) = refs
            npi = p_ref.shape[0]
            y = jnp.concatenate([p_ref[k] for k in range(npi)], axis=1)
            y = jax.nn.relu(y + b_ref[...])
            mu = st_ref[0:1, :] * (1.0 / N)
            var = st_ref[1:2, :] * (1.0 / N) - mu * mu
            z = (y - mu) * lax.rsqrt(var + 1e-5) * g_ref[...] + be_ref[...]
        else:
            (z_ref, w_ref, vs_ref, vd_ref,
             h_ref, es_ref, ed_ref, bt_ref, mx_scr) = refs
            z = z_ref[...]
        i = pl.program_id(0)
        hb = jnp.dot(z, w_ref[...], preferred_element_type=jnp.float32)
        for k in range(ng):
            h_ref[k] = hb[:, k * 64:(k + 1) * 64]
        es = jnp.dot(z, vs_ref[...], preferred_element_type=jnp.float32)
        ed = jnp.dot(z, vd_ref[...], preferred_element_type=jnp.float32)
        es_ref[...] = es
        ed_ref[...] = ed
        ms = jnp.max(es, axis=0)[None, :]
        md = jnp.max(ed, axis=0)[None, :]

        @pl.when(i == 0)
        def _():
            mx_scr[0:1, :] = ms
            mx_scr[1:2, :] = md

        @pl.when(i > 0)
        def _():
            mx_scr[0:1, :] = jnp.maximum(mx_scr[0:1, :], ms)
            mx_scr[1:2, :] = jnp.maximum(mx_scr[1:2, :], md)

        @pl.when(i == NBLK - 1)
        def _():
            bt = jax.nn.relu(mx_scr[0:1, :] + mx_scr[1:2, :])
            bt_ref[...] = jnp.concatenate([bt, bt, bt, bt], axis=1)

    return body


def _head_kernel(z, w, vs, vd, cout):
    ng = 4 * cout // 64
    din = z.shape[1]
    return pl.pallas_call(
        _head_body(ng, False),
        grid=(NBLK,),
        in_specs=[
            pl.BlockSpec((NB, din), lambda i: (i, 0)),
            pl.BlockSpec((din, 4 * cout), lambda i: (0, 0)),
            pl.BlockSpec((din, 4), lambda i: (0, 0)),
            pl.BlockSpec((din, 4), lambda i: (0, 0)),
        ],
        out_specs=[
            pl.BlockSpec((ng, NB, 64), lambda i: (0, i, 0)),
            pl.BlockSpec((NB, 4), lambda i: (i, 0)),
            pl.BlockSpec((NB, 4), lambda i: (i, 0)),
            pl.BlockSpec((1, 16), lambda i: (0, 0)),
        ],
        out_shape=[
            jax.ShapeDtypeStruct((ng, N, 64), jnp.float32),
            jax.ShapeDtypeStruct((N, 4), jnp.float32),
            jax.ShapeDtypeStruct((N, 4), jnp.float32),
            jax.ShapeDtypeStruct((1, 16), jnp.float32),
        ],
        scratch_shapes=[pltpu.VMEM((2, 4), jnp.float32)],
    )(z, w, vs, vd)


def _bn_head_kernel(parts, b, st, g, be, w, vs, vd, cout):
    npi, _, chp = parts.shape
    c = npi * chp
    ng = 4 * cout // 64
    return pl.pallas_call(
        _head_body(ng, True),
        grid=(NBLK,),
        in_specs=[
            pl.BlockSpec((npi, NB, chp), lambda i: (0, i, 0)),
            pl.BlockSpec((1, c), lambda i: (0, 0)),
            pl.BlockSpec((2, c), lambda i: (0, 0)),
            pl.BlockSpec((1, c), lambda i: (0, 0)),
            pl.BlockSpec((1, c), lambda i: (0, 0)),
            pl.BlockSpec((c, 4 * cout), lambda i: (0, 0)),
            pl.BlockSpec((c, 4), lambda i: (0, 0)),
            pl.BlockSpec((c, 4), lambda i: (0, 0)),
        ],
        out_specs=[
            pl.BlockSpec((ng, NB, 64), lambda i: (0, i, 0)),
            pl.BlockSpec((NB, 4), lambda i: (i, 0)),
            pl.BlockSpec((NB, 4), lambda i: (i, 0)),
            pl.BlockSpec((1, 16), lambda i: (0, 0)),
        ],
        out_shape=[
            jax.ShapeDtypeStruct((ng, N, 64), jnp.float32),
            jax.ShapeDtypeStruct((N, 4), jnp.float32),
            jax.ShapeDtypeStruct((N, 4), jnp.float32),
            jax.ShapeDtypeStruct((1, 16), jnp.float32),
        ],
        scratch_shapes=[pltpu.VMEM((2, 4), jnp.float32)],
    )(parts, b, st, g, be, w, vs, vd)


def _stats_kernel(parts, b):
    npi, _, chp = parts.shape
    c = npi * chp

    def body(p_ref, b_ref, st_ref, acc):
        i = pl.program_id(0)
        y = jnp.concatenate([p_ref[k] for k in range(npi)], axis=1)
        y = jax.nn.relu(y + b_ref[...])
        s = jnp.sum(y, axis=0)[None, :]
        s2 = jnp.sum(y * y, axis=0)[None, :]

        @pl.when(i == 0)
        def _():
            acc[0:1, :] = s
            acc[1:2, :] = s2

        @pl.when(i > 0)
        def _():
            acc[0:1, :] = acc[0:1, :] + s
            acc[1:2, :] = acc[1:2, :] + s2

        @pl.when(i == NBLK - 1)
        def _():
            st_ref[...] = acc[...]

    return pl.pallas_call(
        body,
        grid=(NBLK,),
        in_specs=[
            pl.BlockSpec((npi, NB, chp), lambda i: (0, i, 0)),
            pl.BlockSpec((1, c), lambda i: (0, 0)),
        ],
        out_specs=pl.BlockSpec((2, c), lambda i: (0, 0)),
        out_shape=jax.ShapeDtypeStruct((2, c), jnp.float32),
        scratch_shapes=[pltpu.VMEM((2, c), jnp.float32)],
    )(parts, b)


def _tail_kernel(parts, b, st, g, be, batch3d, wp1, bp1, wp2, bp2):
    npi, _, chp = parts.shape
    c = npi * chp

    def body(p_ref, b_ref, st_ref, g_ref, be_ref, bat_ref, wp1_ref, bp1_ref,
             wp2_ref, bp2_ref, out_ref, pooled, cnt):
        i = pl.program_id(0)
        y = jnp.concatenate([p_ref[k] for k in range(npi)], axis=1)
        y = jax.nn.relu(y + b_ref[...])
        mu = st_ref[0:1, :] * (1.0 / N)
        var = st_ref[1:2, :] * (1.0 / N) - mu * mu
        z = (y - mu) * lax.rsqrt(var + 1e-5) * g_ref[...] + be_ref[...]
        bb = bat_ref[0]
        oh = (lax.broadcasted_iota(jnp.int32, (G, NB), 0) == bb).astype(
            jnp.float32)
        ps = jnp.dot(oh, z, preferred_element_type=jnp.float32)
        cs = jnp.dot(oh, jnp.ones((NB, 8), jnp.float32),
                     preferred_element_type=jnp.float32)

        @pl.when(i == 0)
        def _():
            pooled[...] = ps
            cnt[...] = cs

        @pl.when(i > 0)
        def _():
            pooled[...] = pooled[...] + ps
            cnt[...] = cnt[...] + cs

        @pl.when(i == NBLK - 1)
        def _():
            pm = pooled[...] / jnp.clip(cnt[...][:, 0:1], 1.0)
            zp = jax.nn.relu(
                jnp.dot(pm, wp1_ref[...], preferred_element_type=jnp.float32)
                + bp1_ref[...])
            out_ref[...] = (
                jnp.dot(zp, wp2_ref[...], preferred_element_type=jnp.float32)
                + bp2_ref[...])

    return pl.pallas_call(
        body,
        grid=(NBLK,),
        in_specs=[
            pl.BlockSpec((npi, NB, chp), lambda i: (0, i, 0)),
            pl.BlockSpec((1, c), lambda i: (0, 0)),
            pl.BlockSpec((2, c), lambda i: (0, 0)),
            pl.BlockSpec((1, c), lambda i: (0, 0)),
            pl.BlockSpec((1, c), lambda i: (0, 0)),
            pl.BlockSpec((1, 1, NB), lambda i: (i, 0, 0)),
            pl.BlockSpec((c, 16), lambda i: (0, 0)),
            pl.BlockSpec((1, 16), lambda i: (0, 0)),
            pl.BlockSpec((16, 8), lambda i: (0, 0)),
            pl.BlockSpec((1, 8), lambda i: (0, 0)),
        ],
        out_specs=pl.BlockSpec((G, 8), lambda i: (0, 0)),
        out_shape=jax.ShapeDtypeStruct((G, 8), jnp.float32),
        scratch_shapes=[
            pltpu.VMEM((G, c), jnp.float32),
            pltpu.VMEM((G, 8), jnp.float32),
        ],
    )(parts, b, st, g, be, batch3d, wp1, bp1, wp2, bp2)


@functools.lru_cache(maxsize=None)
def _sc_gat(npass):
    k1 = 512
    k3 = 256
    nc1 = EPT // k1
    nc3 = EPT // k3
    mesh = plsc.VectorSubcoreMesh(core_axis_name="c", subcore_axis_name="s")

    def body(src_h, dst_h, es_h, ed_h, bt_h, hsp_h, zo_h, zd_h,
             outp_h, denp_h, exf_h,
             srci1, dsti1, esb, edb, exb4, srci3, dsti3, didx3,
             exb3, denb, alphab, hbuf, msgb, btv, den_sh, out_sh, sem):
        cid = lax.axis_index("c")
        sid = lax.axis_index("s")
        nb0 = sid * NPT
        pltpu.sync_copy(zd_h.at[pl.ds(nb0, NPT)], den_sh.at[pl.ds(nb0, NPT)])
        pltpu.sync_copy(zo_h.at[pl.ds(nb0, NPT)], out_sh.at[pl.ds(nb0, NPT)])
        pltpu.sync_copy(bt_h, btv)
        iota = lax.iota(jnp.int32, 16)
        rq = lax.shift_right_logical(iota, 2)
        cq = lax.bitwise_and(iota, 3)
        plsc.subcore_barrier()

        btvec = btv[...]
        ebase = sid * EPT

        def p1_chunk(ci, _):
            b = ebase + ci * k1
            pltpu.sync_copy(src_h.at[pl.ds(b, k1)], srci1)
            pltpu.sync_copy(dst_h.at[pl.ds(b, k1)], dsti1)
            pltpu.async_copy(es_h.at[srci1], esb, sem).wait()
            pltpu.async_copy(ed_h.at[dsti1], edb, sem).wait()

            def grp(p, _):
                r = rq + p * 4
                ev = (plsc.load_gather(esb, (r, cq))
                      + plsc.load_gather(edb, (r, cq)))
                ev = jnp.where(ev >= 0.0, ev, 0.2 * ev)
                exv = jnp.exp(ev - btvec)
                plsc.store_scatter(exb4, (r, cq), exv)
                return 0

            lax.fori_loop(0, k1 // 4, grp, 0)
            pltpu.sync_copy(exb4, exf_h.at[pl.ds(cid * E + b, k1)])
            pltpu.sync_copy(exb4, den_sh.at[dsti1], add=True)
            return 0

        lax.fori_loop(0, nc1, p1_chunk, 0)
        plsc.subcore_barrier()
        pltpu.sync_copy(den_sh.at[pl.ds(nb0, NPT)],
                        denp_h.at[pl.ds(cid * N + nb0, NPT)])
        plsc.subcore_barrier()

        for q in range(npass):

            def p3_chunk(ci, _):
                b = ebase + ci * k3
                pltpu.sync_copy(src_h.at[pl.ds(b, k3)], srci3)
                pltpu.sync_copy(dst_h.at[pl.ds(b, k3)], dsti3)
                hoff = (cid * npass + q) * N
                doff = cid * N

                def addoff(p, _):
                    s = pl.ds(p * 16, 16)
                    srci3[s] = srci3[s] + hoff
                    didx3[s] = dsti3[s] + doff
                    return 0

                lax.fori_loop(0, k3 // 16, addoff, 0)
                pltpu.sync_copy(exf_h.at[pl.ds(cid * E + b, k3)], exb3)
                pltpu.async_copy(denp_h.at[didx3], denb, sem).wait()
                pltpu.async_copy(hsp_h.at[srci3], hbuf, sem).wait()

                def agrp(p, _):
                    r = rq + p * 4
                    exv = plsc.load_gather(exb3, (r, cq))
                    dv = plsc.load_gather(denb, (r, cq))
                    alphab[pl.ds(p * 16, 16)] = exv / (dv + 1e-16)
                    return 0

                lax.fori_loop(0, k3 * 4 // 16, agrp, 0)

                def edge(e2, _):
                    av = alphab[pl.ds(4 * e2, 16)] * 0.25
                    a0 = av[0]
                    a1 = av[1]
                    a2 = av[2]
                    a3 = av[3]
                    v = (a0 * hbuf[e2, pl.ds(0, 16)]
                         + a1 * hbuf[e2, pl.ds(CG, 16)]
                         + a2 * hbuf[e2, pl.ds(2 * CG, 16)]
                         + a3 * hbuf[e2, pl.ds(3 * CG, 16)])
                    msgb[e2, :] = v
                    return 0

                lax.fori_loop(0, k3, edge, 0)
                pltpu.sync_copy(msgb, out_sh.at[dsti3], add=True)
                return 0

            lax.fori_loop(0, nc3, p3_chunk, 0)
            plsc.subcore_barrier()
            pltpu.sync_copy(out_sh.at[pl.ds(nb0, NPT)],
                            outp_h.at[pl.ds((cid * npass + q) * N + nb0,
                                            NPT)])
            if q + 1 < npass:
                plsc.subcore_barrier()
                pltpu.sync_copy(zo_h.at[pl.ds(nb0, NPT)],
                                out_sh.at[pl.ds(nb0, NPT)])
                plsc.subcore_barrier()

    return pl.kernel(
        body,
        compiler_params=pltpu.CompilerParams(use_tc_tiling_on_sc=False,
                                             needs_layout_passes=False),
        out_type=[
            jax.ShapeDtypeStruct((2 * npass * N, CG), jnp.float32),
            jax.ShapeDtypeStruct((2 * N, 4), jnp.float32),
            jax.ShapeDtypeStruct((2 * E, 4), jnp.float32),
        ],
        mesh=mesh,
        scratch_types=[
            pltpu.VMEM((k1,), jnp.int32),
            pltpu.VMEM((k1,), jnp.int32),
            pltpu.VMEM((k1, 4), jnp.float32),
            pltpu.VMEM((k1, 4), jnp.float32),
            pltpu.VMEM((k1, 4), jnp.float32),
            pltpu.VMEM((k3,), jnp.int32),
            pltpu.VMEM((k3,), jnp.int32),
            pltpu.VMEM((k3,), jnp.int32),
            pltpu.VMEM((k3, 4), jnp.float32),
            pltpu.VMEM((k3, 4), jnp.float32),
            pltpu.VMEM((k3 * 4 + 16,), jnp.float32),
            pltpu.VMEM((k3, D), jnp.float32),
            pltpu.VMEM((k3, CG), jnp.float32),
            pltpu.VMEM((16,), jnp.float32),
            pltpu.VMEM_SHARED((N, 4), jnp.float32),
            pltpu.VMEM_SHARED((N, CG), jnp.float32),
            pltpu.SemaphoreType.DMA,
        ],
    )


def _perm(c):
    idx = []
    for gidx in range(c // CG):
        for h in range(H):
            for cc in range(CG):
                idx.append(h * c + gidx * CG + cc)
    return jnp.asarray(idx, jnp.int32)


def _fold(w, a):
    c = a.shape[1]
    return jnp.einsum("ihc,hc->ih", w.reshape(w.shape[0], H, c), a)


def _layer(src, dst, z_or_parts, wp, vs, vd, cout, zo, zd, bn_args=None):
    ng = 4 * cout // 64
    if bn_args is None:
        h, es, ed, bt = _head_kernel(z_or_parts, wp, vs, vd, cout)
    else:
        b, st, g, be = bn_args
        h, es, ed, bt = _bn_head_kernel(z_or_parts, b, st, g, be, wp, vs, vd,
                                        cout)
    o, _dn, _xf = _sc_gat(ng // 2)(src, dst, es, ed, bt.reshape(16),
                                   h.reshape(ng * N, 64), zo, zd)
    return o.reshape(ng, N, CG)


def kernel(x, edge_index, batch, W1, a_src1, a_dst1, b1, g1, be1,
           W2, a_src2, a_dst2, b2, g2, be2,
           W3, a_src3, a_dst3, b3, g3, be3,
           Wp1, bp1, Wp2, bp2):
    src = edge_index[0].astype(jnp.int32)
    dst = edge_index[1].astype(jnp.int32)

    wp1_ = W1[:, _perm(64)]
    wp2_ = W2[:, _perm(64)]
    wp3_ = W3[:, _perm(32)]
    vs1, vd1 = _fold(W1, a_src1), _fold(W1, a_dst1)
    vs2, vd2 = _fold(W2, a_src2), _fold(W2, a_dst2)
    vs3, vd3 = _fold(W3, a_src3), _fold(W3, a_dst3)

    zo = jnp.zeros((N, CG), jnp.float32)
    zd = jnp.zeros((N, 4), jnp.float32)

    p1 = _layer(src, dst, x, wp1_, vs1, vd1, 64, zo, zd)
    st1 = _stats_kernel(p1, b1.reshape(1, 64))
    p2 = _layer(src, dst, p1, wp2_, vs2, vd2, 64, zo, zd,
                (b1.reshape(1, 64), st1, g1.reshape(1, 64),
                 be1.reshape(1, 64)))
    st2 = _stats_kernel(p2, b2.reshape(1, 64))
    p3 = _layer(src, dst, p2, wp3_, vs3, vd3, 32, zo, zd,
                (b2.reshape(1, 64), st2, g2.reshape(1, 64),
                 be2.reshape(1, 64)))
    st3 = _stats_kernel(p3, b3.reshape(1, 32))

    batch3d = batch.astype(jnp.int32).reshape(NBLK, 1, NB)
    wp2p = jnp.pad(Wp2, ((0, 0), (0, 7)))
    bp2p = jnp.pad(bp2, (0, 7))
    out = _tail_kernel(p3, b3.reshape(1, 32), st3, g3.reshape(1, 32),
                       be3.reshape(1, 32), batch3d, Wp1,
                       bp1.reshape(1, 16), wp2p, bp2p.reshape(1, 8))
    return out[:, :1]
